# baseline (speedup 1.0000x reference)
import functools

import jax
import jax.numpy as jnp
from jax.experimental import pallas as pl


def _mlp_block_kernel(s_ref, x_ref, w1_ref, b1_ref, w2_ref, b2_ref,
                      w3_ref, b3_ref, w4_ref, b4_ref, out_ref,
                      *, blocks_per_mol: int, n_exp: int):
    t = pl.program_id(0)

    @pl.when(t == 0)
    def _():
        out_ref[...] = jnp.zeros_like(out_ref)

    x = x_ref[...]
    s = s_ref[0, 0, :]
    acc = jnp.zeros((x.shape[0], 1), jnp.float32)
    for e in range(n_exp):
        h = jnp.maximum(
            jax.lax.dot_general(x, w1_ref[e], (((1,), (0,)), ((), ())),
                                preferred_element_type=jnp.float32)
            + b1_ref[e][None, :], 0.0)
        h = jnp.maximum(
            jax.lax.dot_general(h, w2_ref[e], (((1,), (0,)), ((), ())),
                                preferred_element_type=jnp.float32)
            + b2_ref[e][None, :], 0.0)
        h = jnp.maximum(
            jax.lax.dot_general(h, w3_ref[e], (((1,), (0,)), ((), ())),
                                preferred_element_type=jnp.float32)
            + b3_ref[e][None, :], 0.0)
        y = jnp.sum(h * w4_ref[e][None, :, 0], axis=1, keepdims=True) \
            + b4_ref[e][0]
        acc = jnp.where(s[:, None] == e, y, acc)

    partial = jnp.sum(acc)
    mol = t // blocks_per_mol
    rows = jax.lax.broadcasted_iota(jnp.int32, out_ref.shape, 0)
    out_ref[...] += jnp.where(rows == mol, partial, 0.0)


def kernel(species, input, W1, b1, W2, b2, W3, b3, W4, b4):
    B, A = species.shape
    D = input.shape[-1]
    E = W1.shape[0]
    N = B * A
    T = 256
    n_blocks = N // T
    blocks_per_mol = A // T

    s_flat = species.reshape(-1).astype(jnp.int32).reshape(n_blocks, 1, T)
    x_flat = input.reshape(N, D)

    body = functools.partial(_mlp_block_kernel,
                             blocks_per_mol=blocks_per_mol, n_exp=E)
    whole = lambda shape: pl.BlockSpec(shape, lambda t: (0,) * len(shape))
    out = pl.pallas_call(
        body,
        grid=(n_blocks,),
        in_specs=[
            pl.BlockSpec((1, 1, T), lambda t: (t, 0, 0)),
            pl.BlockSpec((T, D), lambda t: (t, 0)),
            whole(W1.shape), whole(b1.shape),
            whole(W2.shape), whole(b2.shape),
            whole(W3.shape), whole(b3.shape),
            whole(W4.shape), whole(b4.shape),
        ],
        out_specs=pl.BlockSpec((B, 1), lambda t: (0, 0)),
        out_shape=jax.ShapeDtypeStruct((B, 1), jnp.float32),
    )(s_flat, x_flat, W1, b1, W2, b2, W3, b3, W4, b4)
    return out
